# baseline (speedup 1.0000x reference)
import functools

import jax
import jax.numpy as jnp
from jax import lax
from jax.experimental import pallas as pl
from jax.experimental.pallas import tpu as pltpu
from jax.experimental.pallas import tpu_sc as plsc


_CB = 8192


def _pack_body(w_ref, out_ref):
    w = w_ref[...]
    parts = []
    for q in range(4):
        t = w[:, q * 2048:(q + 1) * 2048].T
        lo = t[:, :32].astype(jnp.bfloat16)
        hi = t[:, 32:].astype(jnp.bfloat16)
        lo_u = lax.bitcast_convert_type(lo, jnp.uint16).astype(jnp.uint32)
        hi_u = lax.bitcast_convert_type(hi, jnp.uint16).astype(jnp.uint32)
        parts.append(lo_u | (hi_u << 16))
    out_ref[...] = lax.bitcast_convert_type(
        jnp.concatenate(parts, axis=1), jnp.float32)


def _pack_wt(weights):
    emb, vocab = weights.shape
    nblk = pl.cdiv(vocab, _CB)
    rows = nblk * (_CB // 4)
    return pl.pallas_call(
        _pack_body,
        grid=(nblk,),
        in_specs=[pl.BlockSpec((emb, _CB), lambda i: (0, i))],
        out_specs=pl.BlockSpec((_CB // 4, 128), lambda i: (i, 0)),
        out_shape=jax.ShapeDtypeStruct((rows, 128), jnp.float32),
    )(weights)


def _egather_body(cw_ref, emb_any, out_ref, sem):
    nb = out_ref.shape[0]

    def it(j, _):
        for t in range(8):
            i = j * 8 + t
            idx = cw_ref[i]
            pltpu.make_async_copy(emb_any.at[pl.ds(idx, 1)],
                                  out_ref.at[pl.ds(i, 1)], sem).start()
        pltpu.make_async_copy(emb_any.at[pl.ds(0, 8)],
                              out_ref.at[pl.ds(0, 8)], sem).wait()
        return _

    lax.fori_loop(0, nb // 8, it, None)


def _e_gather_tc(emb_table, cw):
    b = cw.shape[0]
    d = emb_table.shape[1]
    return pl.pallas_call(
        _egather_body,
        grid_spec=pltpu.PrefetchScalarGridSpec(
            num_scalar_prefetch=1,
            grid=(1,),
            in_specs=[pl.BlockSpec(memory_space=pl.ANY)],
            out_specs=pl.BlockSpec((b, d), lambda i, *_: (0, 0)),
            scratch_shapes=[pltpu.SemaphoreType.DMA],
        ),
        out_shape=jax.ShapeDtypeStruct((b, d), jnp.float32),
    )(cw, emb_table)


def _sc_gather_dot(e_rows, tbl, ctx_flat):
    info = plsc.get_sparse_core_info()
    nc, ns = info.num_cores, info.num_subcores
    nw = nc * ns
    b, d = e_rows.shape
    p = ctx_flat.shape[0]
    ch = 160
    bpc = ch // 20
    bpw = b // nw
    ppw = p // nw
    n_chunks = ppw // ch
    assert ppw % ch == 0 and n_chunks % 2 == 0 and ppw // bpw == 20

    mesh = plsc.VectorSubcoreMesh(core_axis_name="c", subcore_axis_name="s")

    @functools.partial(
        pl.kernel,
        mesh=mesh,
        compiler_params=pltpu.CompilerParams(needs_layout_passes=False),
        out_type=jax.ShapeDtypeStruct((p,), jnp.float32),
        scratch_types=[
            pltpu.VMEM((ch,), jnp.int32), pltpu.VMEM((ch,), jnp.int32),
            pltpu.VMEM((ch,), jnp.int32), pltpu.VMEM((ch,), jnp.int32),
            pltpu.VMEM((ch,), jnp.int32), pltpu.VMEM((ch,), jnp.int32),
            pltpu.VMEM((ch, 128), jnp.float32),
            pltpu.VMEM((ch, 128), jnp.float32),
            pltpu.VMEM((bpw, d), jnp.float32),
            pltpu.VMEM((ch,), jnp.float32),
            pltpu.SemaphoreType.DMA, pltpu.SemaphoreType.DMA,
        ],
    )
    def k(e_hbm, tbl_hbm, ctx_hbm, dots_out,
          iba, ibb, rma, rmb, qfa, qfb, rowsa, rowsb, erows_v,
          dots_v, gsema, gsemb):
        wid = lax.axis_index("s") * nc + lax.axis_index("c")
        base = wid * bpw
        pltpu.sync_copy(e_hbm.at[pl.ds(base, bpw)], erows_v)
        wbase = wid * ppw
        lanes = lax.iota(jnp.int32, 16)

        def _remap_and_gather(ci, ibuf, rowm, qoff, rows, gsem):
            off = ctx_hbm.at[pl.ds(wbase + ci * ch, ch)]
            pltpu.sync_copy(off, ibuf)
            for t in range(ch // 16):
                sl = pl.ds(t * 16, 16)
                c = ibuf[sl]
                rowm[sl] = ((c >> 13) << 11) | (c & 2047)
                qoff[sl] = ((c >> 11) & 3) * 32
            pltpu.async_copy(tbl_hbm.at[rowm], rows, gsem)

        def _wait(rowm, rows, gsem):
            pltpu.make_async_copy(tbl_hbm.at[rowm], rows, gsem).wait()

        def _compute(ci, rows, qoff):
            ev = None
            qv = None
            res = jnp.zeros((16,), jnp.float32)
            for r in range(ch):
                if r % 16 == 0:
                    qv = qoff[pl.ds(r, 16)]
                    res = jnp.zeros((16,), jnp.float32)
                if r % 20 == 0:
                    bl = ci * bpc + (r // 20)
                    ev = [erows_v[bl, pl.ds(kq * 16, 16)]
                          for kq in range(4)]
                qs = qv[r % 16]
                woff = pl.multiple_of(qs, 32)
                w0 = rows[r, pl.ds(woff, 16)]
                w1 = rows[r, pl.ds(woff + 16, 16)]
                w0i = plsc.bitcast(w0, jnp.int32)
                w1i = plsc.bitcast(w1, jnp.int32)
                lo0 = plsc.bitcast(w0i << 16, jnp.float32)
                lo1 = plsc.bitcast(w1i << 16, jnp.float32)
                hi0 = plsc.bitcast(w0i & jnp.int32(-65536), jnp.float32)
                hi1 = plsc.bitcast(w1i & jnp.int32(-65536), jnp.float32)
                acc = lo0 * ev[0] + lo1 * ev[1] + hi0 * ev[2] + hi1 * ev[3]
                s = jnp.sum(acc)
                res = jnp.where(lanes == (r % 16), s, res)
                if r % 16 == 15:
                    dots_v[pl.ds(r - 15, 16)] = res
            pltpu.sync_copy(dots_v,
                            dots_out.at[pl.ds(wbase + ci * ch, ch)])

        _remap_and_gather(0, iba, rma, qfa, rowsa, gsema)

        def _body(jj, _):
            ca = jj * 2
            _wait(rma, rowsa, gsema)
            _remap_and_gather(ca + 1, ibb, rmb, qfb, rowsb, gsemb)
            _compute(ca, rowsa, qfa)
            _wait(rmb, rowsb, gsemb)

            @pl.when(jj < n_chunks // 2 - 1)
            def _():
                _remap_and_gather(ca + 2, iba, rma, qfa, rowsa, gsema)

            _compute(ca + 1, rowsb, qfb)
            return _

        lax.fori_loop(0, n_chunks // 2, _body, None)

    return k(e_rows, tbl, ctx_flat)


def _logsig_body(s_ref, out_ref):
    s = s_ref[...]
    out_ref[...] = jnp.minimum(s, 0.0) - jnp.log1p(jnp.exp(-jnp.abs(s)))


def _logsig(dots):
    b, o = dots.shape
    bb = 4096
    return pl.pallas_call(
        _logsig_body,
        grid=(b // bb,),
        in_specs=[pl.BlockSpec((bb, o), lambda i: (i, 0))],
        out_specs=pl.BlockSpec((bb, o), lambda i: (i, 0)),
        out_shape=jax.ShapeDtypeStruct((b, o), jnp.float32),
    )(dots)


def kernel(center_word, context_words, emb_table, weights):
    b, o = context_words.shape
    cw = center_word.astype(jnp.int32)
    ctx = context_words.astype(jnp.int32)
    tbl = _pack_wt(weights)
    e_rows = _e_gather_tc(emb_table, cw)
    dots = _sc_gather_dot(e_rows, tbl, ctx.reshape(-1))
    out = _logsig(dots.reshape(b, o))
    true_y = jnp.zeros(b, dtype=jnp.int32)
    return (out, true_y)


# speedup vs baseline: 2.1800x; 2.1800x over previous
import functools

import jax
import jax.numpy as jnp
from jax import lax
from jax.experimental import pallas as pl
from jax.experimental.pallas import tpu as pltpu
from jax.experimental.pallas import tpu_sc as plsc


_CB = 8192


def _pack_body(w_ref, out_ref):
    w = w_ref[...]
    parts = []
    for q in range(4):
        t = w[:, q * 2048:(q + 1) * 2048].T
        lo = t[:, :32].astype(jnp.bfloat16)
        hi = t[:, 32:].astype(jnp.bfloat16)
        lo_u = lax.bitcast_convert_type(lo, jnp.uint16).astype(jnp.uint32)
        hi_u = lax.bitcast_convert_type(hi, jnp.uint16).astype(jnp.uint32)
        parts.append(lo_u | (hi_u << 16))
    out_ref[...] = lax.bitcast_convert_type(
        jnp.concatenate(parts, axis=1), jnp.float32)


def _pack_wt(weights):
    emb, vocab = weights.shape
    nblk = pl.cdiv(vocab, _CB)
    rows = nblk * (_CB // 4)
    return pl.pallas_call(
        _pack_body,
        grid=(nblk,),
        in_specs=[pl.BlockSpec((emb, _CB), lambda i: (0, i))],
        out_specs=pl.BlockSpec((_CB // 4, 128), lambda i: (i, 0)),
        out_shape=jax.ShapeDtypeStruct((rows, 128), jnp.float32),
    )(weights)


def _pack_emb_body(e_ref, out_ref):
    out_ref[:, :64] = e_ref[:4096, :]
    out_ref[:, 64:] = e_ref[4096:, :]


def _pack_emb(emb_table):
    v, d = emb_table.shape
    nblk = pl.cdiv(v, 8192)
    return pl.pallas_call(
        _pack_emb_body,
        grid=(nblk,),
        in_specs=[pl.BlockSpec((8192, d), lambda i: (i, 0))],
        out_specs=pl.BlockSpec((4096, 2 * d), lambda i: (i, 0)),
        out_shape=jax.ShapeDtypeStruct((nblk * 4096, 2 * d), jnp.float32),
    )(emb_table)


def _sc_gather_dot(etbl, tbl, center_word, ctx_flat):
    info = plsc.get_sparse_core_info()
    nc, ns = info.num_cores, info.num_subcores
    nw = nc * ns
    b = center_word.shape[0]
    d = 64
    p = ctx_flat.shape[0]
    ch = 160
    bpc = ch // 20
    bpw = b // nw
    ppw = p // nw
    n_chunks = ppw // ch
    assert ppw % ch == 0 and n_chunks % 2 == 0 and ppw // bpw == 20

    mesh = plsc.VectorSubcoreMesh(core_axis_name="c", subcore_axis_name="s")

    @functools.partial(
        pl.kernel,
        mesh=mesh,
        compiler_params=pltpu.CompilerParams(needs_layout_passes=False),
        out_type=jax.ShapeDtypeStruct((p,), jnp.float32),
        scratch_types=[
            pltpu.VMEM((ch,), jnp.int32), pltpu.VMEM((ch,), jnp.int32),
            pltpu.VMEM((ch,), jnp.int32), pltpu.VMEM((ch,), jnp.int32),
            pltpu.VMEM((ch,), jnp.int32), pltpu.VMEM((ch,), jnp.int32),
            pltpu.VMEM((ch, 128), jnp.float32),
            pltpu.VMEM((ch, 128), jnp.float32),
            pltpu.VMEM((bpw, 2 * d), jnp.float32),
            pltpu.VMEM((bpw,), jnp.int32),
            pltpu.VMEM((bpw,), jnp.int32),
            pltpu.VMEM((ch,), jnp.float32),
            pltpu.SemaphoreType.DMA, pltpu.SemaphoreType.DMA,
            pltpu.SemaphoreType.DMA,
        ],
    )
    def k(etbl_hbm, tbl_hbm, cw_hbm, ctx_hbm, dots_out,
          iba, ibb, rma, rmb, qfa, qfb, rowsa, rowsb, erows_v, cwm_v,
          hoff_v, dots_v, gsema, gsemb, esem):
        wid = lax.axis_index("s") * nc + lax.axis_index("c")
        base = wid * bpw
        pltpu.sync_copy(cw_hbm.at[pl.ds(base, bpw)], cwm_v)
        for t in range(bpw // 16):
            sl = pl.ds(t * 16, 16)
            c = cwm_v[sl]
            hoff_v[sl] = ((c >> 12) & 1) * 64
            cwm_v[sl] = ((c >> 13) << 12) | (c & 4095)
        pltpu.async_copy(etbl_hbm.at[cwm_v], erows_v, esem).wait()

        wbase = wid * ppw
        lanes = lax.iota(jnp.int32, 16)

        def _remap_and_gather(ci, ibuf, rowm, qoff, rows, gsem):
            off = ctx_hbm.at[pl.ds(wbase + ci * ch, ch)]
            pltpu.sync_copy(off, ibuf)
            for t in range(ch // 16):
                sl = pl.ds(t * 16, 16)
                c = ibuf[sl]
                rowm[sl] = ((c >> 13) << 11) | (c & 2047)
                qoff[sl] = ((c >> 11) & 3) * 32
            pltpu.async_copy(tbl_hbm.at[rowm], rows, gsem)

        def _wait(rowm, rows, gsem):
            pltpu.make_async_copy(tbl_hbm.at[rowm], rows, gsem).wait()

        def _compute(ci, rows, qoff, hv, hlane0):
            ev = None
            qv = None
            res = jnp.zeros((16,), jnp.float32)
            for r in range(ch):
                if r % 16 == 0:
                    qv = qoff[pl.ds(r, 16)]
                    res = jnp.zeros((16,), jnp.float32)
                if r % 20 == 0:
                    bl = ci * bpc + (r // 20)
                    hoff = pl.multiple_of(hv[hlane0 + r // 20], 64)
                    ev = [erows_v[bl, pl.ds(hoff + kq * 16, 16)]
                          for kq in range(4)]
                qs = qv[r % 16]
                woff = pl.multiple_of(qs, 32)
                w0 = rows[r, pl.ds(woff, 16)]
                w1 = rows[r, pl.ds(woff + 16, 16)]
                w0i = plsc.bitcast(w0, jnp.int32)
                w1i = plsc.bitcast(w1, jnp.int32)
                lo0 = plsc.bitcast(w0i << 16, jnp.float32)
                lo1 = plsc.bitcast(w1i << 16, jnp.float32)
                hi0 = plsc.bitcast(w0i & jnp.int32(-65536), jnp.float32)
                hi1 = plsc.bitcast(w1i & jnp.int32(-65536), jnp.float32)
                acc = lo0 * ev[0] + lo1 * ev[1] + hi0 * ev[2] + hi1 * ev[3]
                s = jnp.sum(acc)
                res = jnp.where(lanes == (r % 16), s, res)
                if r % 16 == 15:
                    dots_v[pl.ds(r - 15, 16)] = res
            pltpu.sync_copy(dots_v,
                            dots_out.at[pl.ds(wbase + ci * ch, ch)])

        _remap_and_gather(0, iba, rma, qfa, rowsa, gsema)

        def _body(jj, _):
            ca = jj * 2
            hv = hoff_v[pl.ds(jj * 16, 16)]
            _wait(rma, rowsa, gsema)
            _remap_and_gather(ca + 1, ibb, rmb, qfb, rowsb, gsemb)
            _compute(ca, rowsa, qfa, hv, 0)
            _wait(rmb, rowsb, gsemb)

            @pl.when(jj < n_chunks // 2 - 1)
            def _():
                _remap_and_gather(ca + 2, iba, rma, qfa, rowsa, gsema)

            _compute(ca + 1, rowsb, qfb, hv, bpc)
            return _

        lax.fori_loop(0, n_chunks // 2, _body, None)

    return k(etbl, tbl, center_word, ctx_flat)


def _logsig_body(s_ref, out_ref):
    s = s_ref[...]
    out_ref[...] = jnp.minimum(s, 0.0) - jnp.log1p(jnp.exp(-jnp.abs(s)))


def _logsig(dots):
    b, o = dots.shape
    bb = 4096
    return pl.pallas_call(
        _logsig_body,
        grid=(b // bb,),
        in_specs=[pl.BlockSpec((bb, o), lambda i: (i, 0))],
        out_specs=pl.BlockSpec((bb, o), lambda i: (i, 0)),
        out_shape=jax.ShapeDtypeStruct((b, o), jnp.float32),
    )(dots)


def kernel(center_word, context_words, emb_table, weights):
    b, o = context_words.shape
    cw = center_word.astype(jnp.int32)
    ctx = context_words.astype(jnp.int32)
    tbl = _pack_wt(weights)
    etbl = _pack_emb(emb_table)
    dots = _sc_gather_dot(etbl, tbl, cw, ctx.reshape(-1))
    out = _logsig(dots.reshape(b, o))
    true_y = jnp.zeros(b, dtype=jnp.int32)
    return (out, true_y)


# speedup vs baseline: 2.2269x; 1.0215x over previous
import functools

import jax
import jax.numpy as jnp
from jax import lax
from jax.experimental import pallas as pl
from jax.experimental.pallas import tpu as pltpu
from jax.experimental.pallas import tpu_sc as plsc


_CB = 8192


def _pack_body(w_ref, out_ref):
    w = w_ref[...]
    parts = []
    for q in range(4):
        t = w[:, q * 2048:(q + 1) * 2048].T
        lo = t[:, :32].astype(jnp.bfloat16)
        hi = t[:, 32:].astype(jnp.bfloat16)
        lo_u = lax.bitcast_convert_type(lo, jnp.uint16).astype(jnp.uint32)
        hi_u = lax.bitcast_convert_type(hi, jnp.uint16).astype(jnp.uint32)
        parts.append(lo_u | (hi_u << 16))
    out_ref[...] = lax.bitcast_convert_type(
        jnp.concatenate(parts, axis=1), jnp.float32)


def _pack_wt(weights):
    emb, vocab = weights.shape
    nblk = pl.cdiv(vocab, _CB)
    rows = nblk * (_CB // 4)
    return pl.pallas_call(
        _pack_body,
        grid=(nblk,),
        in_specs=[pl.BlockSpec((emb, _CB), lambda i: (0, i))],
        out_specs=pl.BlockSpec((_CB // 4, 128), lambda i: (i, 0)),
        out_shape=jax.ShapeDtypeStruct((rows, 128), jnp.float32),
    )(weights)


_EB = 32


def _egather_body(cw_ref, emb_any, out_ref, sema, semb):
    nb = out_ref.shape[0] // _EB

    def _issue(bi, sem):
        b0 = bi * _EB
        for t in range(_EB):
            i = b0 + t
            pltpu.make_async_copy(emb_any.at[pl.ds(cw_ref[i], 1)],
                                  out_ref.at[pl.ds(i, 1)], sem).start()

    def _drain(sem):
        pltpu.make_async_copy(emb_any.at[pl.ds(0, _EB)],
                              out_ref.at[pl.ds(0, _EB)], sem).wait()

    _issue(0, sema)

    def it(k, _):
        _issue(2 * k + 1, semb)
        _drain(sema)

        @pl.when(k < nb // 2 - 1)
        def _():
            _issue(2 * k + 2, sema)

        _drain(semb)
        return _

    lax.fori_loop(0, nb // 2, it, None)


def _e_gather_tc(emb_table, cw):
    b = cw.shape[0]
    d = emb_table.shape[1]
    return pl.pallas_call(
        _egather_body,
        grid_spec=pltpu.PrefetchScalarGridSpec(
            num_scalar_prefetch=1,
            grid=(1,),
            in_specs=[pl.BlockSpec(memory_space=pl.ANY)],
            out_specs=pl.BlockSpec((b, d), lambda i, *_: (0, 0)),
            scratch_shapes=[pltpu.SemaphoreType.DMA,
                            pltpu.SemaphoreType.DMA],
        ),
        out_shape=jax.ShapeDtypeStruct((b, d), jnp.float32),
    )(cw, emb_table)


def _sc_gather_dot(e_rows, tbl, ctx_flat):
    info = plsc.get_sparse_core_info()
    nc, ns = info.num_cores, info.num_subcores
    nw = nc * ns
    b, d = e_rows.shape
    p = ctx_flat.shape[0]
    ch = 160
    bpc = ch // 20
    bpw = b // nw
    ppw = p // nw
    n_chunks = ppw // ch
    assert ppw % ch == 0 and n_chunks % 2 == 0 and ppw // bpw == 20

    mesh = plsc.VectorSubcoreMesh(core_axis_name="c", subcore_axis_name="s")

    @functools.partial(
        pl.kernel,
        mesh=mesh,
        compiler_params=pltpu.CompilerParams(needs_layout_passes=False),
        out_type=jax.ShapeDtypeStruct((p,), jnp.float32),
        scratch_types=[
            pltpu.VMEM((ch,), jnp.int32), pltpu.VMEM((ch,), jnp.int32),
            pltpu.VMEM((ch,), jnp.int32), pltpu.VMEM((ch,), jnp.int32),
            pltpu.VMEM((ch,), jnp.int32), pltpu.VMEM((ch,), jnp.int32),
            pltpu.VMEM((ch, 128), jnp.float32),
            pltpu.VMEM((ch, 128), jnp.float32),
            pltpu.VMEM((bpw, d), jnp.float32),
            pltpu.VMEM((ch,), jnp.float32),
            pltpu.SemaphoreType.DMA, pltpu.SemaphoreType.DMA,
        ],
    )
    def k(e_hbm, tbl_hbm, ctx_hbm, dots_out,
          iba, ibb, rma, rmb, qfa, qfb, rowsa, rowsb, erows_v,
          dots_v, gsema, gsemb):
        wid = lax.axis_index("s") * nc + lax.axis_index("c")
        base = wid * bpw
        pltpu.sync_copy(e_hbm.at[pl.ds(base, bpw)], erows_v)
        wbase = wid * ppw
        lanes = lax.iota(jnp.int32, 16)

        def _remap_and_gather(ci, ibuf, rowm, qoff, rows, gsem):
            off = ctx_hbm.at[pl.ds(wbase + ci * ch, ch)]
            pltpu.sync_copy(off, ibuf)
            for t in range(ch // 16):
                sl = pl.ds(t * 16, 16)
                c = ibuf[sl]
                rowm[sl] = ((c >> 13) << 11) | (c & 2047)
                qoff[sl] = ((c >> 11) & 3) * 32
            pltpu.async_copy(tbl_hbm.at[rowm], rows, gsem)

        def _wait(rowm, rows, gsem):
            pltpu.make_async_copy(tbl_hbm.at[rowm], rows, gsem).wait()

        def _compute(ci, rows, qoff):
            ev = None
            qv = None
            res = jnp.zeros((16,), jnp.float32)
            for r in range(ch):
                if r % 16 == 0:
                    qv = qoff[pl.ds(r, 16)]
                    res = jnp.zeros((16,), jnp.float32)
                if r % 20 == 0:
                    bl = ci * bpc + (r // 20)
                    ev = [erows_v[bl, pl.ds(kq * 16, 16)]
                          for kq in range(4)]
                qs = qv[r % 16]
                woff = pl.multiple_of(qs, 32)
                w0 = rows[r, pl.ds(woff, 16)]
                w1 = rows[r, pl.ds(woff + 16, 16)]
                w0i = plsc.bitcast(w0, jnp.int32)
                w1i = plsc.bitcast(w1, jnp.int32)
                lo0 = plsc.bitcast(w0i << 16, jnp.float32)
                lo1 = plsc.bitcast(w1i << 16, jnp.float32)
                hi0 = plsc.bitcast(w0i & jnp.int32(-65536), jnp.float32)
                hi1 = plsc.bitcast(w1i & jnp.int32(-65536), jnp.float32)
                acc = lo0 * ev[0] + lo1 * ev[1] + hi0 * ev[2] + hi1 * ev[3]
                s = jnp.sum(acc)
                res = jnp.where(lanes == (r % 16), s, res)
                if r % 16 == 15:
                    dots_v[pl.ds(r - 15, 16)] = res
            pltpu.sync_copy(dots_v,
                            dots_out.at[pl.ds(wbase + ci * ch, ch)])

        _remap_and_gather(0, iba, rma, qfa, rowsa, gsema)

        def _body(jj, _):
            ca = jj * 2
            _wait(rma, rowsa, gsema)
            _remap_and_gather(ca + 1, ibb, rmb, qfb, rowsb, gsemb)
            _compute(ca, rowsa, qfa)
            _wait(rmb, rowsb, gsemb)

            @pl.when(jj < n_chunks // 2 - 1)
            def _():
                _remap_and_gather(ca + 2, iba, rma, qfa, rowsa, gsema)

            _compute(ca + 1, rowsb, qfb)
            return _

        lax.fori_loop(0, n_chunks // 2, _body, None)

    return k(e_rows, tbl, ctx_flat)


def _logsig_body(s_ref, out_ref):
    s = s_ref[...]
    out_ref[...] = jnp.minimum(s, 0.0) - jnp.log1p(jnp.exp(-jnp.abs(s)))


def _logsig(dots):
    b, o = dots.shape
    bb = 4096
    return pl.pallas_call(
        _logsig_body,
        grid=(b // bb,),
        in_specs=[pl.BlockSpec((bb, o), lambda i: (i, 0))],
        out_specs=pl.BlockSpec((bb, o), lambda i: (i, 0)),
        out_shape=jax.ShapeDtypeStruct((b, o), jnp.float32),
    )(dots)


def kernel(center_word, context_words, emb_table, weights):
    b, o = context_words.shape
    cw = center_word.astype(jnp.int32)
    ctx = context_words.astype(jnp.int32)
    tbl = _pack_wt(weights)
    e_rows = _e_gather_tc(emb_table, cw)
    dots = _sc_gather_dot(e_rows, tbl, ctx.reshape(-1))
    out = _logsig(dots.reshape(b, o))
    true_y = jnp.zeros(b, dtype=jnp.int32)
    return (out, true_y)


# speedup vs baseline: 2.8538x; 1.2815x over previous
import functools

import jax
import jax.numpy as jnp
from jax import lax
from jax.experimental import pallas as pl
from jax.experimental.pallas import tpu as pltpu
from jax.experimental.pallas import tpu_sc as plsc


_CB = 8192


def _pack_body(w_ref, out_ref):
    w = w_ref[...]
    parts = []
    for q in range(4):
        t = w[:, q * 2048:(q + 1) * 2048].T
        lo = t[:, :32].astype(jnp.bfloat16)
        hi = t[:, 32:].astype(jnp.bfloat16)
        lo_u = lax.bitcast_convert_type(lo, jnp.uint16).astype(jnp.uint32)
        hi_u = lax.bitcast_convert_type(hi, jnp.uint16).astype(jnp.uint32)
        parts.append(lo_u | (hi_u << 16))
    out_ref[...] = lax.bitcast_convert_type(
        jnp.concatenate(parts, axis=1), jnp.float32)


def _pack_wt(weights):
    emb, vocab = weights.shape
    nblk = pl.cdiv(vocab, _CB)
    rows = nblk * (_CB // 4)
    return pl.pallas_call(
        _pack_body,
        grid=(nblk,),
        in_specs=[pl.BlockSpec((emb, _CB), lambda i: (0, i))],
        out_specs=pl.BlockSpec((_CB // 4, 128), lambda i: (i, 0)),
        out_shape=jax.ShapeDtypeStruct((rows, 128), jnp.float32),
    )(weights)


def _sc_gather_dot(emb_table, tbl, center_word, ctx_flat):
    info = plsc.get_sparse_core_info()
    nc, ns = info.num_cores, info.num_subcores
    nw = nc * ns
    b, d = center_word.shape[0], emb_table.shape[1]
    p = ctx_flat.shape[0]
    ch = 160
    bpc = ch // 20
    bpw = b // nw
    ppw = p // nw
    n_chunks = ppw // ch
    assert ppw % ch == 0 and n_chunks % 2 == 0 and ppw // bpw == 20

    mesh = plsc.VectorSubcoreMesh(core_axis_name="c", subcore_axis_name="s")

    @functools.partial(
        pl.kernel,
        mesh=mesh,
        compiler_params=pltpu.CompilerParams(needs_layout_passes=False),
        out_type=jax.ShapeDtypeStruct((p,), jnp.float32),
        scratch_types=[
            pltpu.VMEM((ch,), jnp.int32), pltpu.VMEM((ch,), jnp.int32),
            pltpu.VMEM((ch,), jnp.int32), pltpu.VMEM((ch,), jnp.int32),
            pltpu.VMEM((ch,), jnp.int32), pltpu.VMEM((ch,), jnp.int32),
            pltpu.VMEM((ch, 128), jnp.float32),
            pltpu.VMEM((ch, 128), jnp.float32),
            pltpu.VMEM((bpw, d), jnp.float32),
            pltpu.VMEM((bpw,), jnp.int32),
            pltpu.VMEM((ch,), jnp.float32),
            pltpu.SemaphoreType.DMA, pltpu.SemaphoreType.DMA,
            pltpu.SemaphoreType.DMA,
        ],
    )
    def k(emb_hbm, tbl_hbm, cw_hbm, ctx_hbm, dots_out,
          iba, ibb, rma, rmb, qfa, qfb, rowsa, rowsb, erows_v, cwidx_v,
          dots_v, gsema, gsemb, esem):
        wid = lax.axis_index("s") * nc + lax.axis_index("c")
        base = wid * bpw
        pltpu.sync_copy(cw_hbm.at[pl.ds(base, bpw)], cwidx_v)
        kk = 16

        def _egather(j0, _):
            jbase = pl.multiple_of(j0 * kk, kk)
            v = cwidx_v[pl.ds(jbase, kk)]
            for j in range(kk):
                r = v[j]
                pltpu.async_copy(emb_hbm.at[pl.ds(r, 1)],
                                 erows_v.at[pl.ds(jbase + j, 1)], esem)
            pltpu.make_async_copy(emb_hbm.at[pl.ds(0, kk)],
                                  erows_v.at[pl.ds(0, kk)], esem).wait()
            return _

        lax.fori_loop(0, bpw // kk, _egather, None)

        wbase = wid * ppw
        lanes = lax.iota(jnp.int32, 16)

        def _remap_and_gather(ci, ibuf, rowm, qoff, rows, gsem):
            off = ctx_hbm.at[pl.ds(wbase + ci * ch, ch)]
            pltpu.sync_copy(off, ibuf)
            for t in range(ch // 16):
                sl = pl.ds(t * 16, 16)
                c = ibuf[sl]
                rowm[sl] = ((c >> 13) << 11) | (c & 2047)
                qoff[sl] = ((c >> 11) & 3) * 32
            pltpu.async_copy(tbl_hbm.at[rowm], rows, gsem)

        def _wait(rowm, rows, gsem):
            pltpu.make_async_copy(tbl_hbm.at[rowm], rows, gsem).wait()

        def _compute(ci, rows, qoff):
            ev = None
            qv = None
            res = jnp.zeros((16,), jnp.float32)
            for r in range(ch):
                if r % 16 == 0:
                    qv = qoff[pl.ds(r, 16)]
                    res = jnp.zeros((16,), jnp.float32)
                if r % 20 == 0:
                    bl = ci * bpc + (r // 20)
                    ev = [erows_v[bl, pl.ds(kq * 16, 16)]
                          for kq in range(4)]
                qs = qv[r % 16]
                woff = pl.multiple_of(qs, 32)
                w0 = rows[r, pl.ds(woff, 16)]
                w1 = rows[r, pl.ds(woff + 16, 16)]
                w0i = plsc.bitcast(w0, jnp.int32)
                w1i = plsc.bitcast(w1, jnp.int32)
                lo0 = plsc.bitcast(w0i << 16, jnp.float32)
                lo1 = plsc.bitcast(w1i << 16, jnp.float32)
                hi0 = plsc.bitcast(w0i & jnp.int32(-65536), jnp.float32)
                hi1 = plsc.bitcast(w1i & jnp.int32(-65536), jnp.float32)
                acc = lo0 * ev[0] + lo1 * ev[1] + hi0 * ev[2] + hi1 * ev[3]
                s = jnp.sum(acc)
                res = jnp.where(lanes == (r % 16), s, res)
                if r % 16 == 15:
                    dots_v[pl.ds(r - 15, 16)] = res
            pltpu.sync_copy(dots_v,
                            dots_out.at[pl.ds(wbase + ci * ch, ch)])

        _remap_and_gather(0, iba, rma, qfa, rowsa, gsema)

        def _body(jj, _):
            ca = jj * 2
            _wait(rma, rowsa, gsema)
            _remap_and_gather(ca + 1, ibb, rmb, qfb, rowsb, gsemb)
            _compute(ca, rowsa, qfa)
            _wait(rmb, rowsb, gsemb)

            @pl.when(jj < n_chunks // 2 - 1)
            def _():
                _remap_and_gather(ca + 2, iba, rma, qfa, rowsa, gsema)

            _compute(ca + 1, rowsb, qfb)
            return _

        lax.fori_loop(0, n_chunks // 2, _body, None)

    return k(emb_table, tbl, center_word, ctx_flat)


def _logsig_body(s_ref, out_ref):
    s = s_ref[...]
    out_ref[...] = jnp.minimum(s, 0.0) - jnp.log1p(jnp.exp(-jnp.abs(s)))


def _logsig(dots):
    b, o = dots.shape
    bb = 4096
    return pl.pallas_call(
        _logsig_body,
        grid=(b // bb,),
        in_specs=[pl.BlockSpec((bb, o), lambda i: (i, 0))],
        out_specs=pl.BlockSpec((bb, o), lambda i: (i, 0)),
        out_shape=jax.ShapeDtypeStruct((b, o), jnp.float32),
    )(dots)


def kernel(center_word, context_words, emb_table, weights):
    b, o = context_words.shape
    cw = center_word.astype(jnp.int32)
    ctx = context_words.astype(jnp.int32)
    tbl = _pack_wt(weights)
    dots = _sc_gather_dot(emb_table, tbl, cw, ctx.reshape(-1))
    out = _logsig(dots.reshape(b, o))
    true_y = jnp.zeros(b, dtype=jnp.int32)
    return (out, true_y)
